# Initial kernel scaffold; baseline (speedup 1.0000x reference)
import jax
import jax.numpy as jnp
from jax.experimental import pallas as pl


def kernel(x, W1, b1, W2, b2):
    raise NotImplementedError("write your pallas kernel here")


# speedup vs baseline: 3.4433x; 3.4433x over previous
import functools

import jax
import jax.numpy as jnp
from jax.experimental import pallas as pl
from jax.experimental.pallas import tpu as pltpu

_BN = 512


def _gating_body(x_ref, w1_ref, b1_ref, w2_ref, b2_ref, out_ref):
    x = x_ref[...]
    h = jax.lax.dot_general(
        x, w1_ref[...], (((1,), (0,)), ((), ())),
        preferred_element_type=jnp.float32,
    )
    h = jnp.maximum(h + b1_ref[...], 0.0)
    logits = jax.lax.dot_general(
        h, w2_ref[...], (((1,), (0,)), ((), ())),
        preferred_element_type=jnp.float32,
    )
    logits = logits + b2_ref[...]

    e_dim = logits.shape[-1]
    lane = jax.lax.broadcasted_iota(jnp.int32, logits.shape, 1)
    big = jnp.int32(e_dim)
    m1 = jnp.max(logits, axis=-1, keepdims=True)
    i1 = jnp.min(jnp.where(logits == m1, lane, big), axis=-1, keepdims=True)
    masked = jnp.where(lane == i1, -jnp.inf, logits)
    m2 = jnp.max(masked, axis=-1, keepdims=True)
    i2 = jnp.min(jnp.where(masked == m2, lane, big), axis=-1, keepdims=True)

    e = jnp.exp(m2 - m1)
    denom = 1.0 + e
    g1 = 1.0 / denom
    g2 = e / denom
    out_ref[...] = (jnp.where(lane == i1, g1, 0.0)
                    + jnp.where(lane == i2, g2, 0.0))


@jax.jit
def kernel(x, W1, b1, W2, b2):
    n, d = x.shape
    h_dim = W1.shape[1]
    e_dim = W2.shape[1]
    b1r = b1.reshape(1, h_dim)
    b2r = b2.reshape(1, e_dim)
    grid = (n // _BN,)
    return pl.pallas_call(
        _gating_body,
        grid=grid,
        in_specs=[
            pl.BlockSpec((_BN, d), lambda i: (i, 0)),
            pl.BlockSpec((d, h_dim), lambda i: (0, 0)),
            pl.BlockSpec((1, h_dim), lambda i: (0, 0)),
            pl.BlockSpec((h_dim, e_dim), lambda i: (0, 0)),
            pl.BlockSpec((1, e_dim), lambda i: (0, 0)),
        ],
        out_specs=pl.BlockSpec((_BN, e_dim), lambda i: (i, 0)),
        out_shape=jax.ShapeDtypeStruct((n, e_dim), jnp.float32),
    )(x, W1, b1r, W2, b2r)


# speedup vs baseline: 3.7061x; 1.0763x over previous
import functools

import jax
import jax.numpy as jnp
from jax.experimental import pallas as pl
from jax.experimental.pallas import tpu as pltpu

_BN = 512


def _gating_body(x_ref, w1_ref, b1_ref, w2_ref, b2_ref, out_ref):
    x = x_ref[...]
    h = jax.lax.dot_general(
        x, w1_ref[...], (((1,), (0,)), ((), ())),
        preferred_element_type=jnp.float32,
    )
    h = jnp.maximum(h + b1_ref[...], 0.0)
    logits = jax.lax.dot_general(
        h, w2_ref[...], (((1,), (0,)), ((), ())),
        preferred_element_type=jnp.float32,
    )
    logits = logits + b2_ref[...]

    m1 = jnp.max(logits, axis=-1, keepdims=True)
    mask1 = logits == m1
    masked = jnp.where(mask1, -jnp.inf, logits)
    m2 = jnp.max(masked, axis=-1, keepdims=True)
    mask2 = masked == m2

    e = jnp.exp(m2 - m1)
    denom = 1.0 + e
    g1 = 1.0 / denom
    g2 = e / denom
    out_ref[...] = (jnp.where(mask1, g1, 0.0)
                    + jnp.where(mask2, g2, 0.0))


@jax.jit
def kernel(x, W1, b1, W2, b2):
    n, d = x.shape
    h_dim = W1.shape[1]
    e_dim = W2.shape[1]
    b1r = b1.reshape(1, h_dim)
    b2r = b2.reshape(1, e_dim)
    grid = (n // _BN,)
    return pl.pallas_call(
        _gating_body,
        grid=grid,
        in_specs=[
            pl.BlockSpec((_BN, d), lambda i: (i, 0)),
            pl.BlockSpec((d, h_dim), lambda i: (0, 0)),
            pl.BlockSpec((1, h_dim), lambda i: (0, 0)),
            pl.BlockSpec((h_dim, e_dim), lambda i: (0, 0)),
            pl.BlockSpec((1, e_dim), lambda i: (0, 0)),
        ],
        out_specs=pl.BlockSpec((_BN, e_dim), lambda i: (i, 0)),
        out_shape=jax.ShapeDtypeStruct((n, e_dim), jnp.float32),
    )(x, W1, b1r, W2, b2r)


# speedup vs baseline: 4.3493x; 1.1735x over previous
import functools

import jax
import jax.numpy as jnp
from jax.experimental import pallas as pl
from jax.experimental.pallas import tpu as pltpu

_BN = 1024


def _gating_body(x_ref, w1_ref, b1_ref, w2_ref, b2_ref, out_ref):
    x = x_ref[...]
    h = jax.lax.dot_general(
        x, w1_ref[...], (((1,), (0,)), ((), ())),
        preferred_element_type=jnp.float32,
    )
    h = jnp.maximum(h + b1_ref[...], 0.0)
    logits = jax.lax.dot_general(
        h, w2_ref[...], (((1,), (0,)), ((), ())),
        preferred_element_type=jnp.float32,
    )
    logits = logits + b2_ref[...]

    m1 = jnp.max(logits, axis=-1, keepdims=True)
    mask1 = logits == m1
    masked = jnp.where(mask1, -jnp.inf, logits)
    m2 = jnp.max(masked, axis=-1, keepdims=True)
    mask2 = masked == m2

    e = jnp.exp(m2 - m1)
    denom = 1.0 + e
    g1 = 1.0 / denom
    g2 = e / denom
    out_ref[...] = (jnp.where(mask1, g1, 0.0)
                    + jnp.where(mask2, g2, 0.0))


@jax.jit
def kernel(x, W1, b1, W2, b2):
    n, d = x.shape
    h_dim = W1.shape[1]
    e_dim = W2.shape[1]
    b1r = b1.reshape(1, h_dim)
    b2r = b2.reshape(1, e_dim)
    grid = (n // _BN,)
    return pl.pallas_call(
        _gating_body,
        grid=grid,
        in_specs=[
            pl.BlockSpec((_BN, d), lambda i: (i, 0)),
            pl.BlockSpec((d, h_dim), lambda i: (0, 0)),
            pl.BlockSpec((1, h_dim), lambda i: (0, 0)),
            pl.BlockSpec((h_dim, e_dim), lambda i: (0, 0)),
            pl.BlockSpec((1, e_dim), lambda i: (0, 0)),
        ],
        out_specs=pl.BlockSpec((_BN, e_dim), lambda i: (i, 0)),
        out_shape=jax.ShapeDtypeStruct((n, e_dim), jnp.float32),
    )(x, W1, b1r, W2, b2r)
